# baseline (speedup 1.0000x reference)
import jax
import jax.numpy as jnp
from jax.experimental import pallas as pl
from jax.experimental.pallas import tpu as pltpu

N = 4096
D_IN = 512
D_OUT = 512

BI = 512
BK = 1024
NI = N // BI
NK = N // BK


def _fused_kernel(x_ref, w_ref, adj_ref, o_ref, h_ref, acc_ref, deg_ref):
    i = pl.program_id(0)
    k = pl.program_id(1)

    @pl.when(i == 0)
    def _build_h():
        h_ref[pl.ds(k * BK, BK), :] = jnp.dot(
            x_ref[...], w_ref[...], preferred_element_type=jnp.float32
        ).astype(jnp.bfloat16)

    a = adj_ref[...]
    partial = jnp.dot(
        a.astype(jnp.bfloat16),
        h_ref[pl.ds(k * BK, BK), :],
        preferred_element_type=jnp.float32,
    )
    rs = jnp.sum(a, axis=1, keepdims=True)

    @pl.when(k == 0)
    def _init():
        acc_ref[...] = partial
        deg_ref[...] = rs

    @pl.when(k != 0)
    def _accum():
        acc_ref[...] += partial
        deg_ref[...] += rs

    @pl.when(k == NK - 1)
    def _emit():
        o_ref[...] = acc_ref[...] / deg_ref[...]


@jax.jit
def kernel(input, adj, W):
    return pl.pallas_call(
        _fused_kernel,
        grid=(NI, NK),
        in_specs=[
            pl.BlockSpec(
                (BK, D_IN),
                lambda i, k: (jnp.where(i == 0, k, NK - 1), 0),
            ),
            pl.BlockSpec((D_IN, D_OUT), lambda i, k: (0, 0)),
            pl.BlockSpec((BI, BK), lambda i, k: (i, k)),
        ],
        out_specs=pl.BlockSpec((BI, D_OUT), lambda i, k: (i, 0)),
        out_shape=jax.ShapeDtypeStruct((N, D_OUT), jnp.float32),
        scratch_shapes=[
            pltpu.VMEM((N, D_OUT), jnp.bfloat16),
            pltpu.VMEM((BI, D_OUT), jnp.float32),
            pltpu.VMEM((BI, 1), jnp.float32),
        ],
        compiler_params=pltpu.CompilerParams(
            dimension_semantics=("arbitrary", "arbitrary"),
        ),
    )(input, W, adj)


# speedup vs baseline: 1.2860x; 1.2860x over previous
import jax
import jax.numpy as jnp
from jax.experimental import pallas as pl
from jax.experimental.pallas import tpu as pltpu

N = 4096
D_IN = 512
D_OUT = 512

BM = 512
BI = 512


def _linear_kernel(x_ref, w_ref, h_ref):
    h_ref[...] = jnp.dot(
        x_ref[...], w_ref[...], preferred_element_type=jnp.float32
    ).astype(jnp.bfloat16)


def _aggr_kernel(adj_ref, h_ref, o_ref):
    a = adj_ref[...]
    deg = jnp.sum(a, axis=1, keepdims=True)
    acc = jnp.dot(
        a.astype(jnp.bfloat16), h_ref[...], preferred_element_type=jnp.float32
    )
    o_ref[...] = acc / deg


@jax.jit
def kernel(input, adj, W):
    h = pl.pallas_call(
        _linear_kernel,
        grid=(N // BM,),
        in_specs=[
            pl.BlockSpec((BM, D_IN), lambda i: (i, 0)),
            pl.BlockSpec((D_IN, D_OUT), lambda i: (0, 0)),
        ],
        out_specs=pl.BlockSpec((BM, D_OUT), lambda i: (i, 0)),
        out_shape=jax.ShapeDtypeStruct((N, D_OUT), jnp.bfloat16),
        compiler_params=pltpu.CompilerParams(
            dimension_semantics=("arbitrary",),
        ),
    )(input, W)

    aggr = pl.pallas_call(
        _aggr_kernel,
        grid=(N // BI,),
        in_specs=[
            pl.BlockSpec((BI, N), lambda i: (i, 0)),
            pl.BlockSpec((N, D_OUT), lambda i: (0, 0)),
        ],
        out_specs=pl.BlockSpec((BI, D_OUT), lambda i: (i, 0)),
        out_shape=jax.ShapeDtypeStruct((N, D_OUT), jnp.float32),
        compiler_params=pltpu.CompilerParams(
            dimension_semantics=("arbitrary",),
        ),
    )(adj, h)

    return aggr


# speedup vs baseline: 1.4695x; 1.1427x over previous
import jax
import jax.numpy as jnp
from jax.experimental import pallas as pl
from jax.experimental.pallas import tpu as pltpu

N = 4096
D_IN = 512
D_OUT = 512

BI = 512
NI = N // BI


def _fused_kernel(x_ref, w_ref, adj_ref, o_ref, h_ref):
    s = pl.program_id(0)

    @pl.when(s == 0)
    def _build_h():
        h_ref[...] = jnp.dot(
            x_ref[...], w_ref[...], preferred_element_type=jnp.float32
        ).astype(jnp.bfloat16)

    @pl.when(s > 0)
    def _aggregate():
        a = adj_ref[...]
        deg = jnp.sum(a, axis=1, keepdims=True)
        acc = jnp.dot(
            a.astype(jnp.bfloat16), h_ref[...],
            preferred_element_type=jnp.float32,
        )
        o_ref[...] = acc / deg


@jax.jit
def kernel(input, adj, W):
    return pl.pallas_call(
        _fused_kernel,
        grid=(NI + 1,),
        in_specs=[
            pl.BlockSpec((N, D_IN), lambda s: (0, 0)),
            pl.BlockSpec((D_IN, D_OUT), lambda s: (0, 0)),
            pl.BlockSpec((BI, N), lambda s: (jnp.maximum(s - 1, 0), 0)),
        ],
        out_specs=pl.BlockSpec(
            (BI, D_OUT), lambda s: (jnp.maximum(s - 1, 0), 0)
        ),
        out_shape=jax.ShapeDtypeStruct((N, D_OUT), jnp.float32),
        scratch_shapes=[
            pltpu.VMEM((N, D_OUT), jnp.bfloat16),
        ],
        compiler_params=pltpu.CompilerParams(
            dimension_semantics=("arbitrary",),
        ),
    )(input, W, adj)
